# Initial kernel scaffold; baseline (speedup 1.0000x reference)
import jax
import jax.numpy as jnp
from jax.experimental import pallas as pl


def kernel(rpn_cls_score, gt_twins):
    raise NotImplementedError("write your pallas kernel here")


# speedup vs baseline: 36.2204x; 36.2204x over previous
import numpy as np
import jax
import jax.numpy as jnp
from jax.experimental import pallas as pl
from jax.experimental.pallas import tpu as pltpu

_FEAT_STRIDE = 8
_SCALES = np.array([2, 4, 5, 6, 8, 9, 10, 12, 14, 16], dtype=np.float64)
_B, _A, _L, _G = 16, 10, 8192, 16
_AP = 16
_TOT = _L * _A

_base = np.array([0.0, _FEAT_STRIDE - 1.0])
_actr = _base[0] + 0.5 * (_base[1] - _base[0])
_alens = (_base[1] - _base[0] + 1.0) * _SCALES
_anchors0 = np.stack([_actr - 0.5 * (_alens - 1.0), _actr + 0.5 * (_alens - 1.0)], axis=1)
_shifts = np.arange(_L) * _FEAT_STRIDE
_all_anchors = (_anchors0[None, :, :] + _shifts[:, None, None]).reshape(-1, 2)
_keep_np = (_all_anchors[:, 0] >= 0) & (_all_anchors[:, 1] < _L * _FEAT_STRIDE)
_inds_inside = np.nonzero(_keep_np)[0]
_NIN = int(_inds_inside.size)


def _pad_rows(x, fill_from_row0=True, fill_value=0.0):
    x = np.asarray(x, np.float32)
    if fill_from_row0:
        pad = np.broadcast_to(x[0:1], (_AP - _A, _L))
    else:
        pad = np.full((_AP - _A, _L), fill_value, np.float32)
    return np.ascontiguousarray(np.concatenate([x, pad], axis=0), dtype=np.float32)


_AST = _pad_rows(_all_anchors[:, 0].reshape(_L, _A).T)
_AEN = _pad_rows(_all_anchors[:, 1].reshape(_L, _A).T)
_INS = _pad_rows(_keep_np.reshape(_L, _A).T, fill_from_row0=False)

_kf, _kb = jax.random.split(jax.random.key(1))
_rf_inside = np.asarray(jax.random.uniform(_kf, (_B, _NIN)))
_rb_inside = np.asarray(jax.random.uniform(_kb, (_B, _NIN)))


def _rank_grid(r_inside):
    dense = np.full((_B, _TOT), np.inf, dtype=np.float32)
    dense[:, _inds_inside] = r_inside
    ranks = np.empty((_B, _TOT), dtype=np.float32)
    ar = np.arange(_TOT, dtype=np.float32)
    for b in range(_B):
        perm = np.argsort(dense[b], kind="stable")
        ranks[b, perm] = ar
    grid = ranks.reshape(_B, _L, _A).transpose(0, 2, 1)
    pad = np.full((_B, _AP - _A, _L), float(_TOT), np.float32)
    return np.ascontiguousarray(np.concatenate([grid, pad], axis=1))


_RF = _rank_grid(_rf_inside)
_RB = _rank_grid(_rb_inside)


def _label_kernel(gs_ref, ge_ref, gl_ref, gc_ref, ast_ref, aen_ref, ins_ref,
                  rf_ref, rb_ref, lab_ref, dx_ref, dl_ref, iw_ref, cnt_ref):
    ast = ast_ref[...]
    aen = aen_ref[...]
    inside = ins_ref[...] > 0.5
    alen = aen - ast + 1.0

    max_ov = jnp.full(ast.shape, -1.0, jnp.float32)
    bglen = jnp.full(ast.shape, 1.0, jnp.float32)
    bgctr = jnp.zeros(ast.shape, jnp.float32)
    anyt = jnp.zeros(ast.shape, jnp.bool_)
    for g in range(_G):
        gs = gs_ref[0, 0, g]
        ge = ge_ref[0, 0, g]
        gl = gl_ref[0, 0, g]
        gc = gc_ref[0, 0, g]
        i_s = jnp.maximum(ast, gs)
        i_e = jnp.minimum(aen, ge)
        il = jnp.maximum(i_e - i_s + 1.0, 0.0)
        ov = il / ((alen + gl) - il)
        ov = jnp.where(gl == 1.0, 0.0, ov)
        gmax = jnp.max(jnp.where(inside, ov, -1.0))
        gmax = jnp.where(gmax == 0.0, 1e-5, gmax)
        anyt = anyt | (ov == gmax)
        upd = ov > max_ov
        max_ov = jnp.where(upd, ov, max_ov)
        bglen = jnp.where(upd, gl, bglen)
        bgctr = jnp.where(upd, gc, bgctr)

    labels = jnp.where(max_ov < 0.3, 0.0, -1.0)
    labels = jnp.where(anyt, 1.0, labels)
    labels = jnp.where(max_ov >= 0.7, 1.0, labels)
    labels = jnp.where(inside, labels, -1.0)

    def _take_first(mask, ranks, quota):
        maskf = jnp.where(mask, 1.0, 0.0)
        m = jnp.minimum(jnp.sum(maskf), quota)

        def body(_, lohi):
            lo, hi = lohi
            mid = (lo + hi) // 2
            c = jnp.sum(jnp.where(ranks <= mid.astype(jnp.float32), maskf, 0.0))
            pred = c >= m
            return (jnp.where(pred, lo, mid), jnp.where(pred, mid, hi))

        _, hi = jax.lax.fori_loop(0, 17, body, (jnp.int32(-1), jnp.int32(_TOT - 1)))
        keep = mask & (ranks <= hi.astype(jnp.float32))
        return keep, m

    rf = rf_ref[0]
    rb = rb_ref[0]
    fg = labels == 1.0
    fkeep, mfg = _take_first(fg, rf, 128.0)
    labels = jnp.where(fg & jnp.logical_not(fkeep), -1.0, labels)
    bg = labels == 0.0
    bkeep, mbg = _take_first(bg, rb, 256.0 - mfg)
    labels = jnp.where(bg & jnp.logical_not(bkeep), -1.0, labels)

    lab_ref[0] = labels
    iw_ref[0] = jnp.where(labels == 1.0, 1.0, 0.0)
    exctr = ast + 0.5 * alen
    dx = (bgctr - exctr) / alen
    dl = jnp.log(bglen / alen)
    dx_ref[0] = jnp.where(inside, dx, 0.0)
    dl_ref[0] = jnp.where(inside, dl, 0.0)
    cnt_ref[0, 0, 0] = mfg + mbg


def _outw_kernel(cnt_ref, lab_ref, ow_ref):
    pw = 1.0 / cnt_ref[_B - 1, 0, 0]
    lab = lab_ref[0]
    ow_ref[0] = jnp.where((lab == 0.0) | (lab == 1.0), pw, 0.0)


def kernel(rpn_cls_score, gt_twins):
    del rpn_cls_score
    gs = gt_twins[:, None, :, 0]
    ge = gt_twins[:, None, :, 1]
    gl = ge - gs + 1.0
    gc = gs + 0.5 * gl

    bspec_gt = pl.BlockSpec((1, 1, _G), lambda b: (b, 0, 0), memory_space=pltpu.SMEM)
    bspec_const = pl.BlockSpec((_AP, _L), lambda b: (0, 0))
    bspec_bal = pl.BlockSpec((1, _AP, _L), lambda b: (b, 0, 0))
    bspec_cnt = pl.BlockSpec((1, 1, 1), lambda b: (b, 0, 0), memory_space=pltpu.SMEM)

    labels, dx, dl, iw, cnts = pl.pallas_call(
        _label_kernel,
        grid=(_B,),
        in_specs=[bspec_gt] * 4 + [bspec_const] * 3 + [bspec_bal] * 2,
        out_specs=[bspec_bal] * 4 + [bspec_cnt],
        out_shape=[
            jax.ShapeDtypeStruct((_B, _AP, _L), jnp.float32),
            jax.ShapeDtypeStruct((_B, _AP, _L), jnp.float32),
            jax.ShapeDtypeStruct((_B, _AP, _L), jnp.float32),
            jax.ShapeDtypeStruct((_B, _AP, _L), jnp.float32),
            jax.ShapeDtypeStruct((_B, 1, 1), jnp.float32),
        ],
    )(gs, ge, gl, gc, jnp.asarray(_AST), jnp.asarray(_AEN), jnp.asarray(_INS),
      jnp.asarray(_RF), jnp.asarray(_RB))

    ow = pl.pallas_call(
        _outw_kernel,
        grid=(_B,),
        in_specs=[pl.BlockSpec((_B, 1, 1), lambda b: (0, 0, 0), memory_space=pltpu.SMEM),
                  bspec_bal],
        out_specs=bspec_bal,
        out_shape=jax.ShapeDtypeStruct((_B, _AP, _L), jnp.float32),
    )(cnts, labels)

    labels_out = labels[:, :_A].reshape(_B, 1, _A * _L, 1, 1)
    tt_out = jnp.stack([dx[:, :_A], dl[:, :_A]], axis=2).reshape(_B, 2 * _A, _L)[:, :, :, None, None]
    in_w_out = jnp.repeat(iw[:, :_A], 2, axis=1)[:, :, :, None, None]
    out_w_out = jnp.repeat(ow[:, :_A], 2, axis=1)[:, :, :, None, None]
    return labels_out, tt_out, in_w_out, out_w_out


# speedup vs baseline: 43.6828x; 1.2060x over previous
import numpy as np
import jax
import jax.numpy as jnp
from jax.experimental import pallas as pl
from jax.experimental.pallas import tpu as pltpu

_FEAT_STRIDE = 8
_SCALES = np.array([2, 4, 5, 6, 8, 9, 10, 12, 14, 16], dtype=np.float64)
_B, _A, _L, _G = 16, 10, 8192, 16
_TOT = _L * _A
_R, _C = 8, _TOT // 8

_base = np.array([0.0, _FEAT_STRIDE - 1.0])
_actr = _base[0] + 0.5 * (_base[1] - _base[0])
_alens = (_base[1] - _base[0] + 1.0) * _SCALES
_anchors0 = np.stack([_actr - 0.5 * (_alens - 1.0), _actr + 0.5 * (_alens - 1.0)], axis=1)
_shifts = np.arange(_L) * _FEAT_STRIDE
_all_anchors = (_anchors0[None, :, :] + _shifts[:, None, None]).reshape(-1, 2)
_keep_np = (_all_anchors[:, 0] >= 0) & (_all_anchors[:, 1] < _L * _FEAT_STRIDE)
_inds_inside = np.nonzero(_keep_np)[0]
_NIN = int(_inds_inside.size)


def _pack(flat):
    return np.ascontiguousarray(
        np.asarray(flat, np.float32).reshape(_L, _A).T.reshape(_R, _C))


_AST = _pack(_all_anchors[:, 0])
_AEN = _pack(_all_anchors[:, 1])
_INS = _pack(_keep_np)

_kf, _kb = jax.random.split(jax.random.key(1))
_rf_inside = np.asarray(jax.random.uniform(_kf, (_B, _NIN)))
_rb_inside = np.asarray(jax.random.uniform(_kb, (_B, _NIN)))


def _rank_grid(r_inside):
    dense = np.full((_B, _TOT), np.inf, dtype=np.float32)
    dense[:, _inds_inside] = r_inside
    ranks = np.empty((_B, _TOT), dtype=np.float32)
    ar = np.arange(_TOT, dtype=np.float32)
    out = np.empty((_B, _R, _C), dtype=np.float32)
    for b in range(_B):
        perm = np.argsort(dense[b], kind="stable")
        ranks[b, perm] = ar
        out[b] = _pack(ranks[b])
    return out


_RF = _rank_grid(_rf_inside)
_RB = _rank_grid(_rb_inside)


def _label_kernel(gs_ref, ge_ref, gl_ref, gc_ref, ast_ref, aen_ref, ins_ref,
                  rf_ref, rb_ref, lab_ref, dx_ref, dl_ref, iw_ref, ow_ref,
                  ne_ref):
    ast = ast_ref[...]
    aen = aen_ref[...]
    inside = ins_ref[...] > 0.5
    alen = aen - ast + 1.0

    max_ov = jnp.full(ast.shape, -1.0, jnp.float32)
    bglen = jnp.full(ast.shape, 1.0, jnp.float32)
    bgctr = jnp.zeros(ast.shape, jnp.float32)
    anyt = jnp.zeros(ast.shape, jnp.bool_)
    for g in range(_G):
        gs = gs_ref[0, 0, g]
        ge = ge_ref[0, 0, g]
        gl = gl_ref[0, 0, g]
        gc = gc_ref[0, 0, g]
        i_s = jnp.maximum(ast, gs)
        i_e = jnp.minimum(aen, ge)
        il = jnp.maximum(i_e - i_s + 1.0, 0.0)
        ov = il / ((alen + gl) - il)
        ov = jnp.where(gl == 1.0, 0.0, ov)
        gmax = jnp.max(jnp.where(inside, ov, -1.0))
        gmax = jnp.where(gmax == 0.0, 1e-5, gmax)
        anyt = anyt | (ov == gmax)
        upd = ov > max_ov
        max_ov = jnp.where(upd, ov, max_ov)
        bglen = jnp.where(upd, gl, bglen)
        bgctr = jnp.where(upd, gc, bgctr)

    labels = jnp.where(max_ov < 0.3, 0.0, -1.0)
    labels = jnp.where(anyt, 1.0, labels)
    labels = jnp.where(max_ov >= 0.7, 1.0, labels)
    labels = jnp.where(inside, labels, -1.0)

    def _take_first(mask, ranks, quota):
        maskf = jnp.where(mask, 1.0, 0.0)
        m = jnp.minimum(jnp.sum(maskf), quota)

        def body(_, lohi):
            lo, hi = lohi
            mid = (lo + hi) // 2
            c = jnp.sum(jnp.where(ranks <= mid.astype(jnp.float32), maskf, 0.0))
            pred = c >= m
            return (jnp.where(pred, lo, mid), jnp.where(pred, mid, hi))

        _, hi = jax.lax.fori_loop(0, 17, body, (jnp.int32(-1), jnp.int32(_TOT - 1)))
        keep = mask & (ranks <= hi.astype(jnp.float32))
        return keep, m

    rf = rf_ref[0]
    rb = rb_ref[0]
    fg = labels == 1.0
    fkeep, mfg = _take_first(fg, rf, 128.0)
    labels = jnp.where(fg & jnp.logical_not(fkeep), -1.0, labels)
    bg = labels == 0.0
    bkeep, mbg = _take_first(bg, rb, 256.0 - mfg)
    labels = jnp.where(bg & jnp.logical_not(bkeep), -1.0, labels)

    @pl.when(pl.program_id(0) == 0)
    def _():
        ne_ref[0, 0] = mfg + mbg

    lab_ref[0] = labels
    iw_ref[0] = jnp.where(labels == 1.0, 1.0, 0.0)
    pw = 1.0 / ne_ref[0, 0]
    ow_ref[0] = jnp.where((labels == 0.0) | (labels == 1.0), pw, 0.0)
    exctr = ast + 0.5 * alen
    dx = (bgctr - exctr) / alen
    dl = jnp.log(bglen / alen)
    dx_ref[0] = jnp.where(inside, dx, 0.0)
    dl_ref[0] = jnp.where(inside, dl, 0.0)


def kernel(rpn_cls_score, gt_twins):
    del rpn_cls_score
    gs = gt_twins[:, None, :, 0]
    ge = gt_twins[:, None, :, 1]
    gl = ge - gs + 1.0
    gc = gs + 0.5 * gl

    def _bmap(b):
        return (b + _B - 1) % _B

    bspec_gt = pl.BlockSpec((1, 1, _G), lambda b: (_bmap(b), 0, 0),
                            memory_space=pltpu.SMEM)
    bspec_const = pl.BlockSpec((_R, _C), lambda b: (0, 0))
    bspec_bal = pl.BlockSpec((1, _R, _C), lambda b: (_bmap(b), 0, 0))

    labels, dx, dl, iw, ow = pl.pallas_call(
        _label_kernel,
        grid=(_B,),
        in_specs=[bspec_gt] * 4 + [bspec_const] * 3 + [bspec_bal] * 2,
        out_specs=[bspec_bal] * 5,
        out_shape=[jax.ShapeDtypeStruct((_B, _R, _C), jnp.float32)] * 5,
        scratch_shapes=[pltpu.SMEM((1, 1), jnp.float32)],
    )(gs, ge, gl, gc, jnp.asarray(_AST), jnp.asarray(_AEN), jnp.asarray(_INS),
      jnp.asarray(_RF), jnp.asarray(_RB))

    labels_out = labels.reshape(_B, 1, _A * _L, 1, 1)
    dx = dx.reshape(_B, _A, _L)
    dl = dl.reshape(_B, _A, _L)
    tt_out = jnp.stack([dx, dl], axis=2).reshape(_B, 2 * _A, _L)[:, :, :, None, None]
    in_w_out = jnp.repeat(iw.reshape(_B, _A, _L), 2, axis=1)[:, :, :, None, None]
    out_w_out = jnp.repeat(ow.reshape(_B, _A, _L), 2, axis=1)[:, :, :, None, None]
    return labels_out, tt_out, in_w_out, out_w_out
